# baseline (speedup 1.0000x reference)
import functools

import jax
import jax.numpy as jnp
from jax import lax
from jax.experimental import pallas as pl
from jax.experimental.pallas import tpu as pltpu
from jax.experimental.pallas import tpu_sc as plsc

DIM = 256
TOK = 8192
NC = 2
NS = 16
NW = NC * NS
BPW = TOK // NW


_DIMS = (((1,), (1,)), ((), ()))


def _argmin_body(K, KT, TT, bf16_carry, fused, *refs):
    if fused:
        r_ref, q_ref, cn_ref, cb_ref, idx_ref, rout_ref, mse_ref, acc_ref = refs
        d = r_ref[...] - q_ref[...]
        rout_ref[...] = d
    else:
        r_ref, cn_ref, cb_ref, idx_ref = refs
        d = r_ref[...]
    rn = jnp.sum(d * d, axis=1, keepdims=True)
    if fused:
        t = pl.program_id(0)
        s = jnp.sum(rn, axis=0, keepdims=True)

        @pl.when(t == 0)
        def _():
            acc_ref[...] = s

        @pl.when(t != 0)
        def _():
            acc_ref[...] = acc_ref[...] + s

        mse_ref[...] = acc_ref[...] * (1.0 / (TOK * DIM))
    lhs = (2.0 * d).astype(jnp.bfloat16)
    iota = lax.broadcasted_iota(jnp.int32, (TT, KT), 1)
    best_val = jnp.full((TT, 1), jnp.inf, jnp.float32)
    best_idx = jnp.zeros((TT, 1), jnp.int32)
    for kt in range(K // KT):
        c = cb_ref[pl.ds(kt * KT, KT), :]
        cn = cn_ref[:, pl.ds(kt * KT, KT)]
        mm = lax.dot_general(lhs, c, _DIMS,
                             preferred_element_type=jnp.float32)
        scores = (rn - mm) + cn
        minv = jnp.min(scores, axis=1, keepdims=True)
        lidx = jnp.min(jnp.where(scores == minv, iota, jnp.int32(2**30)),
                       axis=1, keepdims=True) + (kt * KT)
        better = (minv < best_val) | ((minv == best_val) & (lidx < best_idx))
        best_idx = jnp.where(better, lidx, best_idx)
        best_val = jnp.where(better, minv, best_val)
        if bf16_carry:
            best_val = best_val.astype(jnp.bfloat16).astype(jnp.float32)
    idx_ref[...] = best_idx


def _vq_argmin(r, q, cb, cn):
    K = cb.shape[0]
    TT = 4096
    KT = min(K, 2048)
    fused = q is not None
    big = pl.BlockSpec((TT, DIM), lambda t: (t, 0))
    col = pl.BlockSpec((TT, 1), lambda t: (t, 0))
    one = pl.BlockSpec((1, 1), lambda t: (0, 0))
    in_specs = ([big] + ([big] if fused else [])
                + [pl.BlockSpec((1, K), lambda t: (0, 0)),
                   pl.BlockSpec((K, DIM), lambda t: (0, 0))])
    out_specs = [col]
    out_shape = [jax.ShapeDtypeStruct((TOK, 1), jnp.int32)]
    if fused:
        out_specs += [big, one]
        out_shape += [jax.ShapeDtypeStruct((TOK, DIM), jnp.float32),
                      jax.ShapeDtypeStruct((1, 1), jnp.float32)]
    args = (r,) + ((q,) if fused else ()) + (cn, cb)
    out = pl.pallas_call(
        functools.partial(_argmin_body, K, KT, TT, K > 2048, fused),
        grid=(TOK // TT,),
        in_specs=in_specs,
        out_specs=out_specs,
        out_shape=out_shape,
        scratch_shapes=[pltpu.VMEM((1, 1), jnp.float32)] if fused else [],
    )(*args)
    return out if fused else out[0]


def _make_sc_gather(K):
    mesh = plsc.VectorSubcoreMesh(core_axis_name="c", subcore_axis_name="s")

    @functools.partial(
        pl.kernel, mesh=mesh,
        out_type=jax.ShapeDtypeStruct((TOK, DIM), jnp.float32),
        scratch_types=[pltpu.VMEM((BPW,), jnp.int32),
                       pltpu.VMEM((BPW, DIM), jnp.float32),
                       pltpu.SemaphoreType.DMA],
    )
    def k(cb_hbm, idx_hbm, q_hbm, idx_v, rows_v, sem):
        cid = lax.axis_index("c")
        sid = lax.axis_index("s")
        wid = sid * NC + cid
        base = wid * BPW
        pltpu.sync_copy(idx_hbm.at[pl.ds(base, BPW)], idx_v)
        pltpu.async_copy(cb_hbm.at[idx_v], rows_v, sem).wait()
        pltpu.sync_copy(rows_v, q_hbm.at[pl.ds(base, BPW)])

    return k


def _make_sc_bincount(K):
    mesh = plsc.VectorSubcoreMesh(core_axis_name="c", subcore_axis_name="s")

    @functools.partial(
        pl.kernel, mesh=mesh,
        out_type=jax.ShapeDtypeStruct((NC, K), jnp.float32),
        scratch_types=[pltpu.VMEM((BPW,), jnp.int32),
                       pltpu.VMEM((K,), jnp.float32),
                       pltpu.VMEM((BPW,), jnp.float32),
                       pltpu.VMEM_SHARED((K,), jnp.float32)],
    )
    def k(idx_hbm, counts_hbm, idx_v, z_v, ones_v, cnt_sh):
        cid = lax.axis_index("c")
        sid = lax.axis_index("s")
        wid = sid * NC + cid
        base = wid * BPW
        pltpu.sync_copy(idx_hbm.at[pl.ds(base, BPW)], idx_v)
        z16 = jnp.zeros((16,), jnp.float32)
        def zb(i, c):
            z_v[pl.ds(i * 16, 16)] = z16
            return c
        lax.fori_loop(0, K // 16, zb, 0)
        o16 = jnp.ones((16,), jnp.float32)
        def ob(i, c):
            ones_v[pl.ds(i * 16, 16)] = o16
            return c
        lax.fori_loop(0, BPW // 16, ob, 0)

        @pl.when(sid == 0)
        def _():
            pltpu.sync_copy(z_v, cnt_sh)
        plsc.subcore_barrier()
        pltpu.sync_copy(ones_v, cnt_sh.at[idx_v], add=True)
        plsc.subcore_barrier()

        @pl.when(sid == 0)
        def _():
            pltpu.sync_copy(cnt_sh, counts_hbm.at[cid])

    return k


def _final_body(r_ref, q_ref, x_ref, mse_ref, qout_ref, acc_ref):
    t = pl.program_id(0)
    d = r_ref[...] - q_ref[...]
    qout_ref[...] = x_ref[...] - d
    s = jnp.sum(d * d, axis=1, keepdims=True)
    s = jnp.sum(s, axis=0, keepdims=True)

    @pl.when(t == 0)
    def _():
        acc_ref[...] = s

    @pl.when(t != 0)
    def _():
        acc_ref[...] = acc_ref[...] + s

    mse_ref[...] = acc_ref[...] * (1.0 / (TOK * DIM))


def _final_resid(r, q, x):
    TT = 4096
    big = pl.BlockSpec((TT, DIM), lambda t: (t, 0))
    one = pl.BlockSpec((1, 1), lambda t: (0, 0))
    return pl.pallas_call(
        _final_body,
        grid=(TOK // TT,),
        in_specs=[big, big, big],
        out_specs=[one, big],
        out_shape=[jax.ShapeDtypeStruct((1, 1), jnp.float32),
                   jax.ShapeDtypeStruct((TOK, DIM), jnp.float32)],
        scratch_shapes=[pltpu.VMEM((1, 1), jnp.float32)],
    )(r, q, x)


def _perp_body(c0_ref, c1_ref, c2_ref, p0_ref, p1_ref, p2_ref):
    for cref, pref in ((c0_ref, p0_ref), (c1_ref, p1_ref), (c2_ref, p2_ref)):
        cnt = jnp.sum(cref[...], axis=0, keepdims=True)
        p = cnt / float(TOK)
        e = jnp.sum(p * jnp.log(p + 1e-10), axis=1, keepdims=True)
        pref[...] = jnp.exp(-e)


def _perplexity(c0, c1, c2):
    full = lambda arr: pl.BlockSpec(arr.shape, lambda: (0,) * arr.ndim)
    one = pl.BlockSpec((1, 1), lambda: (0, 0))
    return pl.pallas_call(
        _perp_body,
        in_specs=[full(c0), full(c1), full(c2)],
        out_specs=[one, one, one],
        out_shape=[jax.ShapeDtypeStruct((1, 1), jnp.float32)] * 3,
    )(c0, c1, c2)


def kernel(x, codebook_0, codebook_1, codebook_2):
    B, N, _ = x.shape
    r0 = x.reshape(TOK, DIM)
    cns = [jnp.sum(cb * cb, axis=1)[None, :]
           for cb in (codebook_0, codebook_1, codebook_2)]

    idx0 = _vq_argmin(r0, None, codebook_0, cns[0])
    q0 = _make_sc_gather(codebook_0.shape[0])(codebook_0, idx0.reshape(TOK))
    cnt0 = _make_sc_bincount(codebook_0.shape[0])(idx0.reshape(TOK))
    idx1, r1, m0 = _vq_argmin(r0, q0, codebook_1, cns[1])
    q1 = _make_sc_gather(codebook_1.shape[0])(codebook_1, idx1.reshape(TOK))
    cnt1 = _make_sc_bincount(codebook_1.shape[0])(idx1.reshape(TOK))
    idx2, r2, m1 = _vq_argmin(r1, q1, codebook_2, cns[2])
    q2 = _make_sc_gather(codebook_2.shape[0])(codebook_2, idx2.reshape(TOK))
    cnt2 = _make_sc_bincount(codebook_2.shape[0])(idx2.reshape(TOK))
    m2, qout = _final_resid(r2, q2, r0)

    p0, p1, p2 = _perplexity(cnt0, cnt1, cnt2)

    quantized_out = qout.reshape(B, N, DIM)
    indices_cat = jnp.concatenate(
        [idx0.reshape(B, 1, N), idx1.reshape(B, 1, N), idx2.reshape(B, 1, N)],
        axis=1)
    mses = jnp.concatenate([m0.reshape(1), m1.reshape(1), m2.reshape(1)])
    loss_cat = mses + 0.25 * mses
    perplexity_cat = jnp.concatenate(
        [p0.reshape(1), p1.reshape(1), p2.reshape(1)])
    return (quantized_out, indices_cat, loss_cat, perplexity_cat, mses)


# speedup vs baseline: 1.2070x; 1.2070x over previous
import functools

import jax
import jax.numpy as jnp
from jax import lax
from jax.experimental import pallas as pl
from jax.experimental.pallas import tpu as pltpu
from jax.experimental.pallas import tpu_sc as plsc

DIM = 256
TOK = 8192
NC = 2
NS = 16
NW = NC * NS
BPW = TOK // NW


_DIMS = (((1,), (1,)), ((), ()))


def _argmin_body(K, KT, TT, bf16_carry, fused, *refs):
    if fused:
        r_ref, q_ref, cn_ref, cb_ref, idx_ref, rout_ref, mse_ref, acc_ref = refs
        d = r_ref[...] - q_ref[...]
        rout_ref[...] = d
    else:
        r_ref, cn_ref, cb_ref, idx_ref = refs
        d = r_ref[...]
    rn = jnp.sum(d * d, axis=1, keepdims=True)
    if fused:
        t = pl.program_id(0)
        s = jnp.sum(rn, axis=0, keepdims=True)

        @pl.when(t == 0)
        def _():
            acc_ref[...] = s

        @pl.when(t != 0)
        def _():
            acc_ref[...] = acc_ref[...] + s

        mse_ref[...] = acc_ref[...] * (1.0 / (TOK * DIM))
    lhs = (2.0 * d).astype(jnp.bfloat16)
    iota = lax.broadcasted_iota(jnp.int32, (TT, KT), 1)
    best_val = jnp.full((TT, 1), jnp.inf, jnp.float32)
    best_idx = jnp.zeros((TT, 1), jnp.int32)
    for kt in range(K // KT):
        c = cb_ref[pl.ds(kt * KT, KT), :]
        cn = cn_ref[:, pl.ds(kt * KT, KT)]
        mm = lax.dot_general(lhs, c, _DIMS,
                             preferred_element_type=jnp.float32)
        scores = (rn - mm) + cn
        minv = jnp.min(scores, axis=1, keepdims=True)
        lidx = jnp.min(jnp.where(scores == minv, iota, jnp.int32(2**30)),
                       axis=1, keepdims=True) + (kt * KT)
        better = (minv < best_val) | ((minv == best_val) & (lidx < best_idx))
        best_idx = jnp.where(better, lidx, best_idx)
        best_val = jnp.where(better, minv, best_val)
        if bf16_carry:
            best_val = best_val.astype(jnp.bfloat16).astype(jnp.float32)
    idx_ref[...] = best_idx


def _vq_argmin(r, q, cb, cn):
    K = cb.shape[0]
    TT = 2048
    KT = min(K, 2048)
    fused = q is not None
    big = pl.BlockSpec((TT, DIM), lambda t: (t, 0))
    col = pl.BlockSpec((TT, 1), lambda t: (t, 0))
    one = pl.BlockSpec((1, 1), lambda t: (0, 0))
    in_specs = ([big] + ([big] if fused else [])
                + [pl.BlockSpec((1, K), lambda t: (0, 0)),
                   pl.BlockSpec((K, DIM), lambda t: (0, 0))])
    out_specs = [col]
    out_shape = [jax.ShapeDtypeStruct((TOK, 1), jnp.int32)]
    if fused:
        out_specs += [big, one]
        out_shape += [jax.ShapeDtypeStruct((TOK, DIM), jnp.float32),
                      jax.ShapeDtypeStruct((1, 1), jnp.float32)]
    args = (r,) + ((q,) if fused else ()) + (cn, cb)
    out = pl.pallas_call(
        functools.partial(_argmin_body, K, KT, TT, K > 2048, fused),
        grid=(TOK // TT,),
        in_specs=in_specs,
        out_specs=out_specs,
        out_shape=out_shape,
        scratch_shapes=[pltpu.VMEM((1, 1), jnp.float32)] if fused else [],
    )(*args)
    return out if fused else out[0]


def _make_sc_gather(K):
    mesh = plsc.VectorSubcoreMesh(core_axis_name="c", subcore_axis_name="s")

    @functools.partial(
        pl.kernel, mesh=mesh,
        out_type=jax.ShapeDtypeStruct((TOK, DIM), jnp.float32),
        scratch_types=[pltpu.VMEM((BPW,), jnp.int32),
                       pltpu.VMEM((BPW, DIM), jnp.float32),
                       pltpu.SemaphoreType.DMA],
    )
    def k(cb_hbm, idx_hbm, q_hbm, idx_v, rows_v, sem):
        cid = lax.axis_index("c")
        sid = lax.axis_index("s")
        wid = sid * NC + cid
        base = wid * BPW
        pltpu.sync_copy(idx_hbm.at[pl.ds(base, BPW)], idx_v)
        pltpu.async_copy(cb_hbm.at[idx_v], rows_v, sem).wait()
        pltpu.sync_copy(rows_v, q_hbm.at[pl.ds(base, BPW)])

    return k


def _make_sc_bincount(K):
    mesh = plsc.VectorSubcoreMesh(core_axis_name="c", subcore_axis_name="s")

    @functools.partial(
        pl.kernel, mesh=mesh,
        out_type=jax.ShapeDtypeStruct((NC, K), jnp.float32),
        scratch_types=[pltpu.VMEM((BPW,), jnp.int32),
                       pltpu.VMEM((K,), jnp.float32),
                       pltpu.VMEM((BPW,), jnp.float32),
                       pltpu.VMEM_SHARED((K,), jnp.float32)],
    )
    def k(idx_hbm, counts_hbm, idx_v, z_v, ones_v, cnt_sh):
        cid = lax.axis_index("c")
        sid = lax.axis_index("s")
        wid = sid * NC + cid
        base = wid * BPW
        pltpu.sync_copy(idx_hbm.at[pl.ds(base, BPW)], idx_v)
        z16 = jnp.zeros((16,), jnp.float32)
        def zb(i, c):
            z_v[pl.ds(i * 16, 16)] = z16
            return c
        lax.fori_loop(0, K // 16, zb, 0)
        o16 = jnp.ones((16,), jnp.float32)
        def ob(i, c):
            ones_v[pl.ds(i * 16, 16)] = o16
            return c
        lax.fori_loop(0, BPW // 16, ob, 0)

        @pl.when(sid == 0)
        def _():
            pltpu.sync_copy(z_v, cnt_sh)
        plsc.subcore_barrier()
        pltpu.sync_copy(ones_v, cnt_sh.at[idx_v], add=True)
        plsc.subcore_barrier()

        @pl.when(sid == 0)
        def _():
            pltpu.sync_copy(cnt_sh, counts_hbm.at[cid])

    return k


def _final_body(r_ref, q_ref, x_ref, mse_ref, qout_ref, acc_ref):
    t = pl.program_id(0)
    d = r_ref[...] - q_ref[...]
    qout_ref[...] = x_ref[...] - d
    s = jnp.sum(d * d, axis=1, keepdims=True)
    s = jnp.sum(s, axis=0, keepdims=True)

    @pl.when(t == 0)
    def _():
        acc_ref[...] = s

    @pl.when(t != 0)
    def _():
        acc_ref[...] = acc_ref[...] + s

    mse_ref[...] = acc_ref[...] * (1.0 / (TOK * DIM))


def _final_resid(r, q, x):
    TT = 2048
    big = pl.BlockSpec((TT, DIM), lambda t: (t, 0))
    one = pl.BlockSpec((1, 1), lambda t: (0, 0))
    return pl.pallas_call(
        _final_body,
        grid=(TOK // TT,),
        in_specs=[big, big, big],
        out_specs=[one, big],
        out_shape=[jax.ShapeDtypeStruct((1, 1), jnp.float32),
                   jax.ShapeDtypeStruct((TOK, DIM), jnp.float32)],
        scratch_shapes=[pltpu.VMEM((1, 1), jnp.float32)],
    )(r, q, x)


def _perp_body(c0_ref, c1_ref, c2_ref, p0_ref, p1_ref, p2_ref):
    for cref, pref in ((c0_ref, p0_ref), (c1_ref, p1_ref), (c2_ref, p2_ref)):
        cnt = jnp.sum(cref[...], axis=0, keepdims=True)
        p = cnt / float(TOK)
        e = jnp.sum(p * jnp.log(p + 1e-10), axis=1, keepdims=True)
        pref[...] = jnp.exp(-e)


def _perplexity(c0, c1, c2):
    full = lambda arr: pl.BlockSpec(arr.shape, lambda: (0,) * arr.ndim)
    one = pl.BlockSpec((1, 1), lambda: (0, 0))
    return pl.pallas_call(
        _perp_body,
        in_specs=[full(c0), full(c1), full(c2)],
        out_specs=[one, one, one],
        out_shape=[jax.ShapeDtypeStruct((1, 1), jnp.float32)] * 3,
    )(c0, c1, c2)


def kernel(x, codebook_0, codebook_1, codebook_2):
    B, N, _ = x.shape
    r0 = x.reshape(TOK, DIM)
    cns = [jnp.sum(cb * cb, axis=1)[None, :]
           for cb in (codebook_0, codebook_1, codebook_2)]

    idx0 = _vq_argmin(r0, None, codebook_0, cns[0])
    q0 = _make_sc_gather(codebook_0.shape[0])(codebook_0, idx0.reshape(TOK))
    cnt0 = _make_sc_bincount(codebook_0.shape[0])(idx0.reshape(TOK))
    idx1, r1, m0 = _vq_argmin(r0, q0, codebook_1, cns[1])
    q1 = _make_sc_gather(codebook_1.shape[0])(codebook_1, idx1.reshape(TOK))
    cnt1 = _make_sc_bincount(codebook_1.shape[0])(idx1.reshape(TOK))
    idx2, r2, m1 = _vq_argmin(r1, q1, codebook_2, cns[2])
    q2 = _make_sc_gather(codebook_2.shape[0])(codebook_2, idx2.reshape(TOK))
    cnt2 = _make_sc_bincount(codebook_2.shape[0])(idx2.reshape(TOK))
    m2, qout = _final_resid(r2, q2, r0)

    p0, p1, p2 = _perplexity(cnt0, cnt1, cnt2)

    quantized_out = qout.reshape(B, N, DIM)
    indices_cat = jnp.concatenate(
        [idx0.reshape(B, 1, N), idx1.reshape(B, 1, N), idx2.reshape(B, 1, N)],
        axis=1)
    mses = jnp.concatenate([m0.reshape(1), m1.reshape(1), m2.reshape(1)])
    loss_cat = mses + 0.25 * mses
    perplexity_cat = jnp.concatenate(
        [p0.reshape(1), p1.reshape(1), p2.reshape(1)])
    return (quantized_out, indices_cat, loss_cat, perplexity_cat, mses)
